# speedup vs baseline: 1.7558x; 1.7558x over previous
import functools

import jax
import jax.numpy as jnp
from jax import lax
from jax.experimental import pallas as pl
from jax.experimental.pallas import tpu as pltpu
from jax.experimental.pallas import tpu_sc as plsc

HIDDEN = 768

_info = plsc.get_sparse_core_info()
_NC, _NS = _info.num_cores, _info.num_subcores
_NW = _NC * _NS


def _sc_gather(ids_flat, word_table, n_tokens, chunk):
    b_per_w = n_tokens // _NW
    n_chunks = b_per_w // chunk
    mesh = plsc.VectorSubcoreMesh(core_axis_name="c", subcore_axis_name="s")

    @functools.partial(
        pl.kernel,
        mesh=mesh,
        out_type=jax.ShapeDtypeStruct((n_tokens, HIDDEN), jnp.float32),
        scratch_types=[
            pltpu.VMEM((chunk,), jnp.int32),
            pltpu.VMEM((chunk, HIDDEN), jnp.float32),
            pltpu.SemaphoreType.DMA,
        ],
    )
    def gather_kernel(idx_hbm, table_hbm, out_hbm, idx_v, rows_v, sem):
        wid = lax.axis_index("s") * _NC + lax.axis_index("c")
        base = wid * b_per_w
        for c in range(n_chunks):
            off = base + c * chunk
            pltpu.sync_copy(idx_hbm.at[pl.ds(off, chunk)], idx_v)
            pltpu.async_copy(table_hbm.at[idx_v], rows_v, sem).wait()
            pltpu.sync_copy(rows_v, out_hbm.at[pl.ds(off, chunk)])

    return gather_kernel(ids_flat, word_table)


def _tc_ln_kernel(g_ref, tt_ref, pos_ref, type_ref, w_ref, b_ref, o_ref):
    t0 = type_ref[0:1, :]
    t1 = type_ref[1:2, :]
    x = g_ref[...] + pos_ref[...] + t0 + tt_ref[...] * (t1 - t0)
    mu = jnp.mean(x, axis=-1, keepdims=True)
    d = x - mu
    var = jnp.mean(d * d, axis=-1, keepdims=True)
    o_ref[...] = d * lax.rsqrt(var + 1e-12) * w_ref[...] + b_ref[...]


def _tc_ln(gathered, ttf, pos_table, type_table, ln_w, ln_b, n_tokens, blk, seq):
    grid = (n_tokens // blk,)
    pos_blocks = seq // blk
    return pl.pallas_call(
        _tc_ln_kernel,
        grid=grid,
        in_specs=[
            pl.BlockSpec((blk, HIDDEN), lambda i: (i, 0)),
            pl.BlockSpec((blk, 1), lambda i: (i, 0)),
            pl.BlockSpec((blk, HIDDEN), lambda i: (i % pos_blocks, 0)),
            pl.BlockSpec((2, HIDDEN), lambda i: (0, 0)),
            pl.BlockSpec((1, HIDDEN), lambda i: (0, 0)),
            pl.BlockSpec((1, HIDDEN), lambda i: (0, 0)),
        ],
        out_specs=pl.BlockSpec((blk, HIDDEN), lambda i: (i, 0)),
        out_shape=jax.ShapeDtypeStruct((n_tokens, HIDDEN), jnp.float32),
    )(gathered, ttf, pos_table, type_table, ln_w, ln_b)


def kernel(input_ids, token_type_ids, word_table, pos_table, type_table, ln_w, ln_b):
    batch, seq = input_ids.shape
    n_tokens = batch * seq
    ids_flat = input_ids.reshape(-1).astype(jnp.int32)
    ttf = token_type_ids.reshape(-1, 1).astype(jnp.float32)

    gathered = _sc_gather(ids_flat, word_table, n_tokens, chunk=128)
    out = _tc_ln(
        gathered, ttf, pos_table, type_table,
        ln_w.reshape(1, HIDDEN), ln_b.reshape(1, HIDDEN),
        n_tokens, blk=1024, seq=seq,
    )
    return out.reshape(batch, seq, HIDDEN)


# speedup vs baseline: 1.8453x; 1.0510x over previous
import functools

import jax
import jax.numpy as jnp
from jax import lax
from jax.experimental import pallas as pl
from jax.experimental.pallas import tpu as pltpu
from jax.experimental.pallas import tpu_sc as plsc

HIDDEN = 768

_info = plsc.get_sparse_core_info()
_NC, _NS = _info.num_cores, _info.num_subcores
_NW = _NC * _NS


def _sc_gather(ids_flat, word_table, n_tokens, chunk):
    b_per_w = n_tokens // _NW
    n_chunks = b_per_w // chunk
    mesh = plsc.VectorSubcoreMesh(core_axis_name="c", subcore_axis_name="s")

    @functools.partial(
        pl.kernel,
        mesh=mesh,
        out_type=jax.ShapeDtypeStruct((n_tokens, HIDDEN), jnp.float32),
        scratch_types=[
            pltpu.VMEM((2, chunk), jnp.int32),
            pltpu.VMEM((2, chunk, HIDDEN), jnp.float32),
            pltpu.SemaphoreType.DMA,
            pltpu.SemaphoreType.DMA,
        ],
    )
    def gather_kernel(idx_hbm, table_hbm, out_hbm, idx_v, rows_v, sem0, sem1):
        wid = lax.axis_index("s") * _NC + lax.axis_index("c")
        base = wid * b_per_w
        sems = (sem0, sem1)
        copies = [None, None]
        pltpu.sync_copy(idx_hbm.at[pl.ds(base, chunk)], idx_v.at[0])
        copies[0] = pltpu.async_copy(table_hbm.at[idx_v.at[0]], rows_v.at[0], sems[0])
        for c in range(n_chunks):
            b = c % 2
            nb = (c + 1) % 2
            if c + 1 < n_chunks:
                off_n = base + (c + 1) * chunk
                pltpu.sync_copy(idx_hbm.at[pl.ds(off_n, chunk)], idx_v.at[nb])
                copies[nb] = pltpu.async_copy(
                    table_hbm.at[idx_v.at[nb]], rows_v.at[nb], sems[nb])
            copies[b].wait()
            pltpu.sync_copy(rows_v.at[b], out_hbm.at[pl.ds(base + c * chunk, chunk)])

    return gather_kernel(ids_flat, word_table)


def _tc_ln_kernel(g_ref, tt_ref, pos_ref, type_ref, w_ref, b_ref, o_ref):
    t0 = type_ref[0:1, :]
    t1 = type_ref[1:2, :]
    x = g_ref[...] + pos_ref[...] + t0 + tt_ref[...] * (t1 - t0)
    mu = jnp.mean(x, axis=-1, keepdims=True)
    d = x - mu
    var = jnp.mean(d * d, axis=-1, keepdims=True)
    o_ref[...] = d * lax.rsqrt(var + 1e-12) * w_ref[...] + b_ref[...]


def _tc_ln(gathered, ttf, pos_table, type_table, ln_w, ln_b, n_tokens, blk, seq):
    pos_blocks = seq // blk
    batch = n_tokens // seq
    grid = (pos_blocks, batch)
    tok = lambda p, b: (b * pos_blocks + p, 0)
    return pl.pallas_call(
        _tc_ln_kernel,
        grid=grid,
        in_specs=[
            pl.BlockSpec((blk, HIDDEN), tok),
            pl.BlockSpec((blk, 1), tok),
            pl.BlockSpec((blk, HIDDEN), lambda p, b: (p, 0)),
            pl.BlockSpec((2, HIDDEN), lambda p, b: (0, 0)),
            pl.BlockSpec((1, HIDDEN), lambda p, b: (0, 0)),
            pl.BlockSpec((1, HIDDEN), lambda p, b: (0, 0)),
        ],
        out_specs=pl.BlockSpec((blk, HIDDEN), tok),
        out_shape=jax.ShapeDtypeStruct((n_tokens, HIDDEN), jnp.float32),
    )(gathered, ttf, pos_table, type_table, ln_w, ln_b)


def kernel(input_ids, token_type_ids, word_table, pos_table, type_table, ln_w, ln_b):
    batch, seq = input_ids.shape
    n_tokens = batch * seq
    ids_flat = input_ids.reshape(-1).astype(jnp.int32)
    ttf = token_type_ids.reshape(-1, 1).astype(jnp.float32)

    gathered = _sc_gather(ids_flat, word_table, n_tokens, chunk=64)
    out = _tc_ln(
        gathered, ttf, pos_table, type_table,
        ln_w.reshape(1, HIDDEN), ln_b.reshape(1, HIDDEN),
        n_tokens, blk=1024, seq=seq,
    )
    return out.reshape(batch, seq, HIDDEN)
